# baseline (speedup 1.0000x reference)
import jax
import jax.numpy as jnp
from jax.experimental import pallas as pl
from jax.experimental.pallas import tpu as pltpu

B, H, W = 64, 256, 256
HB = 32
NBANDS = H // HB
BF = jnp.bfloat16
F32 = jnp.float32


def _conv_pool_kernel(gp_ref, w1x_ref, w2x_ref, pw_ref, pst_ref,
                      pooled_ref, ss_ref,
                      ohs3, h1b, h13, h2s):
    g = gp_ref[0]
    cid = jax.lax.broadcasted_iota(jnp.int32, (10, 260, 258), 0)
    for j in range(3):
        gs = g[:, j:j + 258]
        ohs3[10 * j:10 * j + 10] = jnp.where(gs[None, :, :] == cid,
                                             1.0, 0.0).astype(BF)
    ohs3[30:31] = jnp.ones((1, 260, 258), BF)

    s1 = jax.lax.dot_general(ohs3[20:30, 2:258, :], pst_ref[...],
                             (((2,), (0,)), ((), ())),
                             preferred_element_type=F32)
    iota_w = jax.lax.broadcasted_iota(jnp.int32, (10, 256), 1).astype(F32)
    cnt = jnp.sum(s1[:, :, 0], axis=1)
    sx = jnp.sum(s1[:, :, 1], axis=1)
    sy = jnp.sum(s1[:, :, 0] * iota_w, axis=1)
    safe = jnp.maximum(cnt, 1.0)
    pos = cnt > 0
    my = jnp.where(pos, sy / safe, 0.0)
    mx = jnp.where(pos, sx / safe, 0.0)
    ss_ref[0, 0] = jnp.stack([cnt, cnt, my, mx], axis=-1)

    for n in range(NBANDS):
        h0 = HB * n
        h1f = sum(
            jax.lax.dot_general(w1x_ref[ky], ohs3[:, h0 + ky:h0 + ky + 34, :],
                                (((1,), (0,)), ((), ())),
                                preferred_element_type=F32)
            for ky in range(3))
        h1b[...] = jnp.maximum(h1f, 0.0).astype(BF)
        for j in range(3):
            h13[33 * j:33 * j + 33] = h1b[:, :, j:j + 256]
        h2f = sum(
            jax.lax.dot_general(w2x_ref[ky], h13[:, ky:ky + 32, :],
                                (((1,), (0,)), ((), ())),
                                preferred_element_type=F32)
            for ky in range(3))
        h2s[...] = jnp.maximum(h2f, 0.0).astype(BF)
        plw = jax.lax.dot_general(h2s[...], pw_ref[...],
                                  (((2,), (0,)), ((), ())),
                                  preferred_element_type=F32)
        pooled_ref[0, n] = jnp.sum(plw, axis=1)


def _heads_kernel(pooled_ref, gfcw_ref, gfcb_ref,
                  ss_ref, sl1w_ref, sl1b_ref, sl2w_ref, sl2b_ref,
                  sl3w_ref, sl3b_ref,
                  rel_ref, rel1w_ref, rel1b_ref, rel2w_ref, rel2b_ref,
                  sz_ref, sz1w_ref, sz1b_ref, sz2w_ref, sz2b_ref,
                  th_ref, th1w_ref, th1b_ref, th2w_ref, th2b_ref,
                  pr1b_ref, pr2w_ref, pr2b_ref, pr3w_ref, pr3b_ref,
                  out_ref):
    def lin(x, w_ref, b_ref):
        return jax.lax.dot_general(x, w_ref[...], (((1,), (0,)), ((), ())),
                                   preferred_element_type=F32) + b_ref[...]

    grid_feat = lin(pooled_ref[...], gfcw_ref, gfcb_ref)
    s = jnp.maximum(lin(ss_ref[...], sl1w_ref, sl1b_ref), 0.0)
    s = jnp.maximum(lin(s, sl2w_ref, sl2b_ref), 0.0)
    slot_feat = lin(s, sl3w_ref, sl3b_ref)
    r = jnp.maximum(lin(rel_ref[...], rel1w_ref, rel1b_ref), 0.0)
    rel_feat = lin(r, rel2w_ref, rel2b_ref)
    z = jnp.maximum(lin(sz_ref[...], sz1w_ref, sz1b_ref), 0.0)
    size_feat = lin(z, sz2w_ref, sz2b_ref)
    t = jnp.maximum(lin(th_ref[...], th1w_ref, th1b_ref), 0.0)
    theme_feat = lin(t, th2w_ref, th2b_ref)
    p = jnp.maximum(pr1b_ref[...], 0.0)
    p = jnp.maximum(lin(p, pr2w_ref, pr2b_ref), 0.0)
    prog_row = lin(p, pr3w_ref, pr3b_ref)
    out_ref[:, 0:128] = grid_feat
    out_ref[:, 128:160] = slot_feat
    out_ref[:, 160:192] = jnp.broadcast_to(prog_row, (B, 32))
    out_ref[:, 192:256] = rel_feat
    out_ref[:, 256:272] = size_feat
    out_ref[:, 272:304] = theme_feat


def _conv_pool_call(gp, w1x, w2x, pw, pst, interpret=False):
    return pl.pallas_call(
        _conv_pool_kernel,
        grid=(B,),
        in_specs=[
            pl.BlockSpec((1, 260, 260), lambda b: (b, 0, 0)),
            pl.BlockSpec((3, 33, 31), lambda b: (0, 0, 0)),
            pl.BlockSpec((3, 64, 99), lambda b: (0, 0, 0)),
            pl.BlockSpec((256, 8), lambda b: (0, 0)),
            pl.BlockSpec((258, 8), lambda b: (0, 0)),
        ],
        out_specs=[
            pl.BlockSpec((1, NBANDS, 64, 8), lambda b: (b, 0, 0, 0)),
            pl.BlockSpec((1, 1, 10, 4), lambda b: (b, 0, 0, 0)),
        ],
        out_shape=[
            jax.ShapeDtypeStruct((B, NBANDS, 64, 8), F32),
            jax.ShapeDtypeStruct((B, 1, 10, 4), F32),
        ],
        scratch_shapes=[
            pltpu.VMEM((31, 260, 258), BF),
            pltpu.VMEM((33, 34, 258), BF),
            pltpu.VMEM((99, 34, 256), BF),
            pltpu.VMEM((64, 32, 256), BF),
        ],
        compiler_params=pltpu.CompilerParams(
            dimension_semantics=("parallel",),
        ),
        name="ctx_conv_pool",
        interpret=interpret,
    )(gp, w1x, w2x, pw, pst)


def _heads_call(args, interpret=False):
    return pl.pallas_call(
        _heads_kernel,
        out_shape=jax.ShapeDtypeStruct((B, 304), F32),
        name="ctx_heads",
        interpret=interpret,
    )(*args)


def _forward_impl(grid, rel_features, size_oracle, theme_priors, p,
                  interpret=False):
    gp = jnp.pad(grid, ((0, 0), (2, 2), (2, 2)), constant_values=10)
    b1 = p['conv1_b']
    w1 = p['conv1_w'].transpose(0, 2, 3, 1)
    w1x = []
    for ky in range(3):
        wk = w1[:, ky].reshape(32, 30)
        bias_col = (b1 if ky == 1 else jnp.zeros_like(b1))[:, None]
        wk = jnp.concatenate([wk, bias_col], axis=1)
        ones_row = jnp.zeros((1, 31), F32).at[0, 30].set(1.0 if ky == 1
                                                         else 0.0)
        w1x.append(jnp.concatenate([wk, ones_row], axis=0))
    w1x = jnp.stack(w1x).astype(BF)
    b2 = p['conv2_b']
    w2 = p['conv2_w'].transpose(0, 2, 3, 1)
    w2x = []
    for ky in range(3):
        wk = w2[:, ky]
        bias = jnp.zeros((64, 3, 1), F32)
        if ky == 1:
            bias = bias.at[:, 1, 0].set(b2)
        w2x.append(jnp.concatenate([wk, bias], axis=2).reshape(64, 99))
    w2x = jnp.stack(w2x).astype(BF)
    wi = jnp.arange(W)
    pw = ((wi[:, None] // HB) == jnp.arange(8)[None, :]).astype(F32) / 1024.0
    pst = jnp.stack([jnp.ones((258,), F32), jnp.arange(258, dtype=F32)] +
                    [jnp.zeros((258,), F32)] * 6, axis=1)
    pooled, ss4 = _conv_pool_call(gp, w1x, w2x, pw.astype(BF), pst.astype(BF),
                                  interpret=interpret)
    pooled_flat = pooled.transpose(0, 2, 1, 3).reshape(B, 64 * 8 * 8)
    ss = ss4.reshape(B, 40)

    def t2(name):
        return p[name].T.astype(F32)

    def b2v(name):
        return p[name][None, :].astype(F32)

    args = (pooled_flat.astype(BF), p['gfc_w'].T.astype(BF), b2v('gfc_b'),
            ss, t2('sl1_w'), b2v('sl1_b'), t2('sl2_w'), b2v('sl2_b'),
            t2('sl3_w'), b2v('sl3_b'),
            rel_features, t2('rel1_w'), b2v('rel1_b'), t2('rel2_w'),
            b2v('rel2_b'),
            size_oracle, t2('sz1_w'), b2v('sz1_b'), t2('sz2_w'), b2v('sz2_b'),
            theme_priors, t2('th1_w'), b2v('th1_b'), t2('th2_w'), b2v('th2_b'),
            b2v('pr1_b'), t2('pr2_w'), b2v('pr2_b'), t2('pr3_w'),
            b2v('pr3_b'))
    return _heads_call(args, interpret=interpret)


def kernel(grid, rel_features, size_oracle, theme_priors, params):
    return _forward_impl(grid, rel_features, size_oracle, theme_priors,
                         params)


# speedup vs baseline: 1.4301x; 1.4301x over previous
import jax
import jax.numpy as jnp
from jax.experimental import pallas as pl
from jax.experimental.pallas import tpu as pltpu

B, H, W = 64, 256, 256
HB = 32
NBANDS = H // HB
BF = jnp.bfloat16
F32 = jnp.float32


def _conv_pool_kernel(gp_ref, w1x_ref, w2x_ref, pw_ref, pst_ref,
                      pooled_ref, ss_ref,
                      ohs3, oh9, h1b, h13, h19, h2s):
    g = gp_ref[0]
    cid = jax.lax.broadcasted_iota(jnp.int32, (10, 260, 258), 0)
    for j in range(3):
        gs = g[:, j:j + 258]
        ohs3[10 * j:10 * j + 10] = jnp.where(gs[None, :, :] == cid,
                                             1.0, 0.0).astype(BF)
    ohs3[30:31] = jnp.ones((1, 260, 258), BF)

    s1 = jax.lax.dot_general(ohs3[20:30, 2:258, :], pst_ref[...],
                             (((2,), (0,)), ((), ())),
                             preferred_element_type=F32)
    iota_w = jax.lax.broadcasted_iota(jnp.int32, (10, 256), 1).astype(F32)
    cnt = jnp.sum(s1[:, :, 0], axis=1)
    sx = jnp.sum(s1[:, :, 1], axis=1)
    sy = jnp.sum(s1[:, :, 0] * iota_w, axis=1)
    safe = jnp.maximum(cnt, 1.0)
    pos = cnt > 0
    my = jnp.where(pos, sy / safe, 0.0)
    mx = jnp.where(pos, sx / safe, 0.0)
    ss_ref[0, 0] = jnp.stack([cnt, cnt, my, mx], axis=-1)

    for n in range(NBANDS):
        h0 = HB * n
        for ky in range(3):
            for kx in range(3):
                t = 3 * ky + kx
                oh9[10 * t:10 * t + 10] = \
                    ohs3[10 * kx:10 * kx + 10, h0 + ky:h0 + ky + 34, :]
        oh9[90:91] = ohs3[30:31, 0:34, :]
        h1f = jax.lax.dot_general(w1x_ref[...], oh9[...],
                                  (((1,), (0,)), ((), ())),
                                  preferred_element_type=F32)
        h1b[...] = jnp.maximum(h1f, 0.0).astype(BF)
        for j in range(3):
            h13[33 * j:33 * j + 33] = h1b[:, :, j:j + 256]
        for ky in range(3):
            h19[99 * ky:99 * ky + 99] = h13[:, ky:ky + 32, :]
        h2f = jax.lax.dot_general(w2x_ref[...], h19[...],
                                  (((1,), (0,)), ((), ())),
                                  preferred_element_type=F32)
        h2s[...] = jnp.maximum(h2f, 0.0).astype(BF)
        plw = jax.lax.dot_general(h2s[...], pw_ref[...],
                                  (((2,), (0,)), ((), ())),
                                  preferred_element_type=F32)
        pooled_ref[0, n] = jnp.sum(plw, axis=1)


def _heads_kernel(pooled_ref, gfcw_ref, gfcb_ref,
                  ss_ref, sl1w_ref, sl1b_ref, sl2w_ref, sl2b_ref,
                  sl3w_ref, sl3b_ref,
                  rel_ref, rel1w_ref, rel1b_ref, rel2w_ref, rel2b_ref,
                  sz_ref, sz1w_ref, sz1b_ref, sz2w_ref, sz2b_ref,
                  th_ref, th1w_ref, th1b_ref, th2w_ref, th2b_ref,
                  pr1b_ref, pr2w_ref, pr2b_ref, pr3w_ref, pr3b_ref,
                  out_ref):
    def lin(x, w_ref, b_ref):
        return jax.lax.dot_general(x, w_ref[...], (((1,), (0,)), ((), ())),
                                   preferred_element_type=F32) + b_ref[...]

    grid_feat = lin(pooled_ref[...], gfcw_ref, gfcb_ref)
    s = jnp.maximum(lin(ss_ref[...], sl1w_ref, sl1b_ref), 0.0)
    s = jnp.maximum(lin(s, sl2w_ref, sl2b_ref), 0.0)
    slot_feat = lin(s, sl3w_ref, sl3b_ref)
    r = jnp.maximum(lin(rel_ref[...], rel1w_ref, rel1b_ref), 0.0)
    rel_feat = lin(r, rel2w_ref, rel2b_ref)
    z = jnp.maximum(lin(sz_ref[...], sz1w_ref, sz1b_ref), 0.0)
    size_feat = lin(z, sz2w_ref, sz2b_ref)
    t = jnp.maximum(lin(th_ref[...], th1w_ref, th1b_ref), 0.0)
    theme_feat = lin(t, th2w_ref, th2b_ref)
    p = jnp.maximum(pr1b_ref[...], 0.0)
    p = jnp.maximum(lin(p, pr2w_ref, pr2b_ref), 0.0)
    prog_row = lin(p, pr3w_ref, pr3b_ref)
    out_ref[:, 0:128] = grid_feat
    out_ref[:, 128:160] = slot_feat
    out_ref[:, 160:192] = jnp.broadcast_to(prog_row, (B, 32))
    out_ref[:, 192:256] = rel_feat
    out_ref[:, 256:272] = size_feat
    out_ref[:, 272:304] = theme_feat


def _conv_pool_call(gp, w1x, w2x, pw, pst, interpret=False):
    return pl.pallas_call(
        _conv_pool_kernel,
        grid=(B,),
        in_specs=[
            pl.BlockSpec((1, 260, 260), lambda b: (b, 0, 0)),
            pl.BlockSpec((33, 91), lambda b: (0, 0)),
            pl.BlockSpec((64, 297), lambda b: (0, 0)),
            pl.BlockSpec((256, 8), lambda b: (0, 0)),
            pl.BlockSpec((258, 8), lambda b: (0, 0)),
        ],
        out_specs=[
            pl.BlockSpec((1, NBANDS, 64, 8), lambda b: (b, 0, 0, 0)),
            pl.BlockSpec((1, 1, 10, 4), lambda b: (b, 0, 0, 0)),
        ],
        out_shape=[
            jax.ShapeDtypeStruct((B, NBANDS, 64, 8), F32),
            jax.ShapeDtypeStruct((B, 1, 10, 4), F32),
        ],
        scratch_shapes=[
            pltpu.VMEM((31, 260, 258), BF),
            pltpu.VMEM((91, 34, 258), BF),
            pltpu.VMEM((33, 34, 258), BF),
            pltpu.VMEM((99, 34, 256), BF),
            pltpu.VMEM((297, 32, 256), BF),
            pltpu.VMEM((64, 32, 256), BF),
        ],
        compiler_params=pltpu.CompilerParams(
            dimension_semantics=("parallel",),
        ),
        name="ctx_conv_pool",
        interpret=interpret,
    )(gp, w1x, w2x, pw, pst)


def _heads_call(args, interpret=False):
    return pl.pallas_call(
        _heads_kernel,
        out_shape=jax.ShapeDtypeStruct((B, 304), F32),
        name="ctx_heads",
        interpret=interpret,
    )(*args)


def _forward_impl(grid, rel_features, size_oracle, theme_priors, p,
                  interpret=False):
    gp = jnp.pad(grid, ((0, 0), (2, 2), (2, 2)), constant_values=10)
    w1 = p['conv1_w'].transpose(0, 2, 3, 1).reshape(32, 90)
    w1x = jnp.concatenate([w1, p['conv1_b'][:, None]], axis=1)
    ones_row = jnp.zeros((1, 91), F32).at[0, 90].set(1.0)
    w1x = jnp.concatenate([w1x, ones_row], axis=0).astype(BF)
    w2 = p['conv2_w'].transpose(0, 2, 3, 1)
    w2x = jnp.concatenate([w2, jnp.zeros((64, 3, 3, 1), F32)], axis=3)
    w2x = w2x.reshape(64, 297).at[:, 164].set(p['conv2_b']).astype(BF)
    wi = jnp.arange(W)
    pw = ((wi[:, None] // HB) == jnp.arange(8)[None, :]).astype(F32) / 1024.0
    pst = jnp.stack([jnp.ones((258,), F32), jnp.arange(258, dtype=F32)] +
                    [jnp.zeros((258,), F32)] * 6, axis=1)
    pooled, ss4 = _conv_pool_call(gp, w1x, w2x, pw.astype(BF), pst.astype(BF),
                                  interpret=interpret)
    pooled_flat = pooled.transpose(0, 2, 1, 3).reshape(B, 64 * 8 * 8)
    ss = ss4.reshape(B, 40)

    def t2(name):
        return p[name].T.astype(F32)

    def b2v(name):
        return p[name][None, :].astype(F32)

    args = (pooled_flat.astype(BF), p['gfc_w'].T.astype(BF), b2v('gfc_b'),
            ss, t2('sl1_w'), b2v('sl1_b'), t2('sl2_w'), b2v('sl2_b'),
            t2('sl3_w'), b2v('sl3_b'),
            rel_features, t2('rel1_w'), b2v('rel1_b'), t2('rel2_w'),
            b2v('rel2_b'),
            size_oracle, t2('sz1_w'), b2v('sz1_b'), t2('sz2_w'), b2v('sz2_b'),
            theme_priors, t2('th1_w'), b2v('th1_b'), t2('th2_w'), b2v('th2_b'),
            b2v('pr1_b'), t2('pr2_w'), b2v('pr2_b'), t2('pr3_w'),
            b2v('pr3_b'))
    return _heads_call(args, interpret=interpret)


def kernel(grid, rel_features, size_oracle, theme_priors, params):
    return _forward_impl(grid, rel_features, size_oracle, theme_priors,
                         params)


# speedup vs baseline: 1.4795x; 1.0346x over previous
import jax
import jax.numpy as jnp
from jax.experimental import pallas as pl
from jax.experimental.pallas import tpu as pltpu

B, H, W = 64, 256, 256
HB = 32
BAND = 64
NBANDS = H // BAND
BF = jnp.bfloat16
F32 = jnp.float32


def _conv_pool_kernel(gp_ref, w1x_ref, w2x_ref, pw_ref, pst_ref,
                      pooled_ref, ss_ref,
                      ohs3, oh9, h1b, h13, h19, h2s):
    g = gp_ref[0]
    cid = jax.lax.broadcasted_iota(jnp.int32, (10, 260, 258), 0)
    for j in range(3):
        gs = g[:, j:j + 258]
        ohs3[10 * j:10 * j + 10] = jnp.where(gs[None, :, :] == cid,
                                             1.0, 0.0).astype(BF)
    ohs3[30:31] = jnp.ones((1, 260, 258), BF)

    s1 = jax.lax.dot_general(ohs3[20:30, 2:258, :], pst_ref[...],
                             (((2,), (0,)), ((), ())),
                             preferred_element_type=F32)
    iota_w = jax.lax.broadcasted_iota(jnp.int32, (10, 256), 1).astype(F32)
    cnt = jnp.sum(s1[:, :, 0], axis=1)
    sx = jnp.sum(s1[:, :, 1], axis=1)
    sy = jnp.sum(s1[:, :, 0] * iota_w, axis=1)
    safe = jnp.maximum(cnt, 1.0)
    pos = cnt > 0
    my = jnp.where(pos, sy / safe, 0.0)
    mx = jnp.where(pos, sx / safe, 0.0)
    ss_ref[0, 0] = jnp.stack([cnt, cnt, my, mx], axis=-1)

    hh = BAND + 2
    for n in range(NBANDS):
        h0 = BAND * n
        for ky in range(3):
            for kx in range(3):
                t = 3 * ky + kx
                oh9[10 * t:10 * t + 10] = \
                    ohs3[10 * kx:10 * kx + 10, h0 + ky:h0 + ky + hh, :]
        oh9[90:91] = ohs3[30:31, 0:hh, :]
        h1f = jax.lax.dot_general(w1x_ref[...], oh9[...],
                                  (((1,), (0,)), ((), ())),
                                  preferred_element_type=F32)
        h1b[...] = jnp.maximum(h1f, 0.0).astype(BF)
        for j in range(3):
            h13[33 * j:33 * j + 33] = h1b[:, :, j:j + 256]
        for ky in range(3):
            h19[99 * ky:99 * ky + 99] = h13[:, ky:ky + BAND, :]
        h2f = jax.lax.dot_general(w2x_ref[...], h19[...],
                                  (((1,), (0,)), ((), ())),
                                  preferred_element_type=F32)
        h2s[...] = jnp.maximum(h2f, 0.0).astype(BF)
        plw = jax.lax.dot_general(h2s[...], pw_ref[...],
                                  (((2,), (0,)), ((), ())),
                                  preferred_element_type=F32)
        pr = jnp.sum(plw.reshape(64, BAND // HB, HB, 8), axis=2)
        for r in range(BAND // HB):
            pooled_ref[0, (BAND // HB) * n + r] = pr[:, r]


def _heads_kernel(pooled_ref, gfcw_ref, gfcb_ref,
                  ss_ref, sl1w_ref, sl1b_ref, sl2w_ref, sl2b_ref,
                  sl3w_ref, sl3b_ref,
                  rel_ref, rel1w_ref, rel1b_ref, rel2w_ref, rel2b_ref,
                  sz_ref, sz1w_ref, sz1b_ref, sz2w_ref, sz2b_ref,
                  th_ref, th1w_ref, th1b_ref, th2w_ref, th2b_ref,
                  pr1b_ref, pr2w_ref, pr2b_ref, pr3w_ref, pr3b_ref,
                  out_ref):
    def lin(x, w_ref, b_ref):
        return jax.lax.dot_general(x, w_ref[...], (((1,), (0,)), ((), ())),
                                   preferred_element_type=F32) + b_ref[...]

    grid_feat = lin(pooled_ref[...], gfcw_ref, gfcb_ref)
    s = jnp.maximum(lin(ss_ref[...], sl1w_ref, sl1b_ref), 0.0)
    s = jnp.maximum(lin(s, sl2w_ref, sl2b_ref), 0.0)
    slot_feat = lin(s, sl3w_ref, sl3b_ref)
    r = jnp.maximum(lin(rel_ref[...], rel1w_ref, rel1b_ref), 0.0)
    rel_feat = lin(r, rel2w_ref, rel2b_ref)
    z = jnp.maximum(lin(sz_ref[...], sz1w_ref, sz1b_ref), 0.0)
    size_feat = lin(z, sz2w_ref, sz2b_ref)
    t = jnp.maximum(lin(th_ref[...], th1w_ref, th1b_ref), 0.0)
    theme_feat = lin(t, th2w_ref, th2b_ref)
    p = jnp.maximum(pr1b_ref[...], 0.0)
    p = jnp.maximum(lin(p, pr2w_ref, pr2b_ref), 0.0)
    prog_row = lin(p, pr3w_ref, pr3b_ref)
    out_ref[:, 0:128] = grid_feat
    out_ref[:, 128:160] = slot_feat
    out_ref[:, 160:192] = jnp.broadcast_to(prog_row, (B, 32))
    out_ref[:, 192:256] = rel_feat
    out_ref[:, 256:272] = size_feat
    out_ref[:, 272:304] = theme_feat


def _conv_pool_call(gp, w1x, w2x, pw, pst, interpret=False):
    return pl.pallas_call(
        _conv_pool_kernel,
        grid=(B,),
        in_specs=[
            pl.BlockSpec((1, 260, 260), lambda b: (b, 0, 0)),
            pl.BlockSpec((33, 91), lambda b: (0, 0)),
            pl.BlockSpec((64, 297), lambda b: (0, 0)),
            pl.BlockSpec((256, 8), lambda b: (0, 0)),
            pl.BlockSpec((258, 8), lambda b: (0, 0)),
        ],
        out_specs=[
            pl.BlockSpec((1, 8, 64, 8), lambda b: (b, 0, 0, 0)),
            pl.BlockSpec((1, 1, 10, 4), lambda b: (b, 0, 0, 0)),
        ],
        out_shape=[
            jax.ShapeDtypeStruct((B, 8, 64, 8), F32),
            jax.ShapeDtypeStruct((B, 1, 10, 4), F32),
        ],
        scratch_shapes=[
            pltpu.VMEM((31, 260, 258), BF),
            pltpu.VMEM((91, BAND + 2, 258), BF),
            pltpu.VMEM((33, BAND + 2, 258), BF),
            pltpu.VMEM((99, BAND + 2, 256), BF),
            pltpu.VMEM((297, BAND, 256), BF),
            pltpu.VMEM((64, BAND, 256), BF),
        ],
        compiler_params=pltpu.CompilerParams(
            dimension_semantics=("parallel",),
        ),
        name="ctx_conv_pool",
        interpret=interpret,
    )(gp, w1x, w2x, pw, pst)


def _heads_call(args, interpret=False):
    return pl.pallas_call(
        _heads_kernel,
        out_shape=jax.ShapeDtypeStruct((B, 304), F32),
        name="ctx_heads",
        interpret=interpret,
    )(*args)


def _forward_impl(grid, rel_features, size_oracle, theme_priors, p,
                  interpret=False):
    gp = jnp.pad(grid, ((0, 0), (2, 2), (2, 2)), constant_values=10)
    w1 = p['conv1_w'].transpose(0, 2, 3, 1).reshape(32, 90)
    w1x = jnp.concatenate([w1, p['conv1_b'][:, None]], axis=1)
    ones_row = jnp.zeros((1, 91), F32).at[0, 90].set(1.0)
    w1x = jnp.concatenate([w1x, ones_row], axis=0).astype(BF)
    w2 = p['conv2_w'].transpose(0, 2, 3, 1)
    w2x = jnp.concatenate([w2, jnp.zeros((64, 3, 3, 1), F32)], axis=3)
    w2x = w2x.reshape(64, 297).at[:, 164].set(p['conv2_b']).astype(BF)
    wi = jnp.arange(W)
    pw = ((wi[:, None] // HB) == jnp.arange(8)[None, :]).astype(F32) / 1024.0
    pst = jnp.stack([jnp.ones((258,), F32), jnp.arange(258, dtype=F32)] +
                    [jnp.zeros((258,), F32)] * 6, axis=1)
    pooled, ss4 = _conv_pool_call(gp, w1x, w2x, pw.astype(BF), pst.astype(BF),
                                  interpret=interpret)
    pooled_flat = pooled.transpose(0, 2, 1, 3).reshape(B, 64 * 8 * 8)
    ss = ss4.reshape(B, 40)

    def t2(name):
        return p[name].T.astype(F32)

    def b2v(name):
        return p[name][None, :].astype(F32)

    args = (pooled_flat.astype(BF), p['gfc_w'].T.astype(BF), b2v('gfc_b'),
            ss, t2('sl1_w'), b2v('sl1_b'), t2('sl2_w'), b2v('sl2_b'),
            t2('sl3_w'), b2v('sl3_b'),
            rel_features, t2('rel1_w'), b2v('rel1_b'), t2('rel2_w'),
            b2v('rel2_b'),
            size_oracle, t2('sz1_w'), b2v('sz1_b'), t2('sz2_w'), b2v('sz2_b'),
            theme_priors, t2('th1_w'), b2v('th1_b'), t2('th2_w'), b2v('th2_b'),
            b2v('pr1_b'), t2('pr2_w'), b2v('pr2_b'), t2('pr3_w'),
            b2v('pr3_b'))
    return _heads_call(args, interpret=interpret)


def kernel(grid, rel_features, size_oracle, theme_priors, params):
    return _forward_impl(grid, rel_features, size_oracle, theme_priors,
                         params)
